# baseline (speedup 1.0000x reference)
import jax
import jax.numpy as jnp
from jax import lax
from jax.experimental import pallas as pl
from jax.experimental.pallas import tpu as pltpu
from jax.experimental.pallas import tpu_sc as plsc

N_CLS = 1024
SEQ_LEN = 77
CTX_DIM = 512
N_CTX = 8
SUFFIX = SEQ_LEN - 1 - N_CTX
N_GATHER = 1 + SUFFIX
IDX_PAD = 72

_info = plsc.get_sparse_core_info()
_NC = _info.num_cores
_NS = _info.num_subcores
_NW = _NC * _NS
_CPW = N_CLS // _NW
_NBUF = 2


def _body(idx_hbm, table_hbm, ctx_hbm, out_hbm,
          idx_v, rows0, rows1, ctx_v, gs0, gs1, ws0, ws1):
    wid = lax.axis_index("s") * _NC + lax.axis_index("c")
    base = wid * _CPW
    rows = (rows0, rows1)
    gsems = (gs0, gs1)
    wsems = (ws0, ws1)

    pltpu.sync_copy(ctx_hbm, ctx_v)
    pltpu.sync_copy(idx_hbm.at[pl.ds(base, _CPW)], idx_v)

    def start_gather(i, b):
        pltpu.async_copy(table_hbm.at[idx_v.at[i]], rows[b], gsems[b])

    def wait_gather(i, b):
        pltpu.make_async_copy(table_hbm.at[idx_v.at[i]], rows[b],
                              gsems[b]).wait()

    def start_writes(c, b):
        pltpu.async_copy(rows[b].at[pl.ds(0, 1)],
                         out_hbm.at[c, pl.ds(0, 1)], wsems[b])
        pltpu.async_copy(ctx_v, out_hbm.at[c, pl.ds(1, N_CTX)], wsems[b])
        pltpu.async_copy(rows[b].at[pl.ds(1, SUFFIX)],
                         out_hbm.at[c, pl.ds(1 + N_CTX, SUFFIX)], wsems[b])

    def wait_writes(c, b):
        pltpu.make_async_copy(rows[b].at[pl.ds(0, 1)],
                              out_hbm.at[c, pl.ds(0, 1)], wsems[b]).wait()
        pltpu.make_async_copy(ctx_v, out_hbm.at[c, pl.ds(1, N_CTX)],
                              wsems[b]).wait()
        pltpu.make_async_copy(rows[b].at[pl.ds(1, SUFFIX)],
                              out_hbm.at[c, pl.ds(1 + N_CTX, SUFFIX)],
                              wsems[b]).wait()

    start_gather(0, 0)
    start_gather(1, 1)

    def step(j, carry):
        for b in range(_NBUF):
            i = j * _NBUF + b
            c = base + i
            wait_gather(i, b)
            start_writes(c, b)

            @pl.when(i + _NBUF < _CPW)
            def _():
                wait_writes(c, b)
                start_gather(i + _NBUF, b)

        return carry

    lax.fori_loop(0, _CPW // _NBUF, step, 0)

    for b in range(_NBUF):
        i = _CPW - _NBUF + b
        wait_writes(base + i, b)


def kernel(tokens, table, ctx):
    idx = jnp.concatenate(
        [tokens[:, :1], tokens[:, 1 + N_CTX:],
         jnp.zeros((N_CLS, IDX_PAD - N_GATHER), jnp.int32)], axis=1)
    f = pl.kernel(
        _body,
        out_type=jax.ShapeDtypeStruct((N_CLS, SEQ_LEN, CTX_DIM), jnp.float32),
        mesh=plsc.VectorSubcoreMesh(core_axis_name="c", subcore_axis_name="s"),
        compiler_params=pltpu.CompilerParams(use_tc_tiling_on_sc=False),
        scratch_types=[
            pltpu.VMEM((_CPW, IDX_PAD), jnp.int32),
            pltpu.VMEM((IDX_PAD, CTX_DIM), jnp.float32),
            pltpu.VMEM((IDX_PAD, CTX_DIM), jnp.float32),
            pltpu.VMEM((N_CTX, CTX_DIM), jnp.float32),
            pltpu.SemaphoreType.DMA,
            pltpu.SemaphoreType.DMA,
            pltpu.SemaphoreType.DMA,
            pltpu.SemaphoreType.DMA,
        ],
    )
    return f(idx, table, ctx)


# speedup vs baseline: 1.5038x; 1.5038x over previous
import jax
import jax.numpy as jnp
from jax import lax
from jax.experimental import pallas as pl
from jax.experimental.pallas import tpu as pltpu
from jax.experimental.pallas import tpu_sc as plsc

N_CLS = 1024
SEQ_LEN = 77
CTX_DIM = 512
N_CTX = 8
SUFFIX = SEQ_LEN - 1 - N_CTX

_info = plsc.get_sparse_core_info()
_NC = _info.num_cores
_NS = _info.num_subcores
_NW = _NC * _NS
_CPW = N_CLS // _NW
_NBUF = 2


def _body(tokens_hbm, table_hbm, ctx_hbm, out_hbm,
          idx0, idx1, rows0, rows1, ctx_v, gs0, gs1):
    wid = lax.axis_index("s") * _NC + lax.axis_index("c")
    base = wid * _CPW
    idxs = (idx0, idx1)
    rows = (rows0, rows1)
    gsems = (gs0, gs1)

    pltpu.sync_copy(ctx_hbm, ctx_v)

    def start_gather(i, b):
        pltpu.sync_copy(tokens_hbm.at[base + i], idxs[b])
        pltpu.async_copy(table_hbm.at[idxs[b]], rows[b], gsems[b])

    def wait_gather(b):
        pltpu.make_async_copy(table_hbm.at[idxs[b]], rows[b],
                              gsems[b]).wait()

    def write_out(i, b):
        c = base + i
        pltpu.sync_copy(rows[b].at[pl.ds(0, 1)], out_hbm.at[c, pl.ds(0, 1)])
        pltpu.sync_copy(ctx_v, out_hbm.at[c, pl.ds(1, N_CTX)])
        pltpu.sync_copy(rows[b].at[pl.ds(1 + N_CTX, SUFFIX)],
                        out_hbm.at[c, pl.ds(1 + N_CTX, SUFFIX)])

    start_gather(0, 0)

    def step(j, carry):
        for b in range(_NBUF):
            i = j * _NBUF + b

            @pl.when(i + 1 < _CPW)
            def _():
                start_gather(i + 1, (b + 1) % _NBUF)

            wait_gather(b)
            write_out(i, b)
        return carry

    lax.fori_loop(0, _CPW // _NBUF, step, 0)


def kernel(tokens, table, ctx):
    f = pl.kernel(
        _body,
        out_type=jax.ShapeDtypeStruct((N_CLS, SEQ_LEN, CTX_DIM), jnp.float32),
        mesh=plsc.VectorSubcoreMesh(core_axis_name="c", subcore_axis_name="s"),
        compiler_params=pltpu.CompilerParams(use_tc_tiling_on_sc=False),
        scratch_types=[
            pltpu.VMEM((SEQ_LEN,), jnp.int32),
            pltpu.VMEM((SEQ_LEN,), jnp.int32),
            pltpu.VMEM((SEQ_LEN, CTX_DIM), jnp.float32),
            pltpu.VMEM((SEQ_LEN, CTX_DIM), jnp.float32),
            pltpu.VMEM((N_CTX, CTX_DIM), jnp.float32),
            pltpu.SemaphoreType.DMA,
            pltpu.SemaphoreType.DMA,
        ],
    )
    return f(tokens, table, ctx)
